# Initial kernel scaffold; baseline (speedup 1.0000x reference)
import jax
import jax.numpy as jnp
from jax.experimental import pallas as pl


def kernel(x, edge_index, edge_norm, W_rgc, W_u, W_i):
    raise NotImplementedError("write your pallas kernel here")


# speedup vs baseline: 4.7337x; 4.7337x over previous
import functools

import jax
import jax.numpy as jnp
from jax import lax
from jax.experimental import pallas as pl
from jax.experimental.pallas import tpu as pltpu
from jax.experimental.pallas import tpu_sc as plsc

NC = 2
NS = 16
NW = NC * NS
CHUNK = 128
LANES = 16


def _bcast_lane(g, l):
  idx = jnp.full((LANES, 1), l, jnp.int32)
  return lax.gather(
      g, idx,
      dimension_numbers=lax.GatherDimensionNumbers(
          offset_dims=(), collapsed_slice_dims=(0,), start_index_map=(0,)),
      slice_sizes=(1,), mode=lax.GatherScatterMode.PROMISE_IN_BOUNDS)


def _sc_segment_sum(x, src_p, dst_p, norm_p, zeros, n_pad, d, nchunk):
  rows_per_tile = n_pad // NS

  mesh = plsc.VectorSubcoreMesh(core_axis_name="c", subcore_axis_name="s")

  @functools.partial(
      pl.kernel,
      out_type=jax.ShapeDtypeStruct((NC, n_pad, d), jnp.float32),
      mesh=mesh,
      scratch_types=[
          pltpu.VMEM((nchunk, CHUNK), jnp.int32),
          pltpu.VMEM((nchunk, CHUNK), jnp.int32),
          pltpu.VMEM((nchunk * CHUNK,), jnp.float32),
          pltpu.VMEM((CHUNK, d), jnp.float32),
          pltpu.VMEM_SHARED((n_pad, d), jnp.float32),
          pltpu.SemaphoreType.DMA,
      ],
  )
  def seg_kernel(x_hbm, src_hbm, dst_hbm, norm_hbm, z_hbm, out_hbm,
                 src_v, dst_v, norm_v, rows_v, agg_sh, sem):
    c = lax.axis_index("c")
    s = lax.axis_index("s")
    wid = c * NS + s

    epw = nchunk * CHUNK
    pltpu.sync_copy(src_hbm.at[wid], src_v)
    pltpu.sync_copy(dst_hbm.at[wid], dst_v)
    pltpu.sync_copy(norm_hbm.at[pl.ds(wid * epw, epw)], norm_v)

    row0 = s * rows_per_tile
    pltpu.sync_copy(z_hbm.at[pl.ds(row0, rows_per_tile)],
                    agg_sh.at[pl.ds(row0, rows_per_tile)])
    plsc.subcore_barrier()

    def chunk_body(j, carry):
      pltpu.async_copy(x_hbm.at[src_v.at[j]], rows_v, sem).wait()

      def group_body(gi, carry2):
        g = norm_v[pl.ds(j * CHUNK + gi * LANES, LANES)]
        for l in range(LANES):
          nb = _bcast_lane(g, l)
          r = gi * LANES + l
          for k in range(d // LANES):
            sl = pl.ds(k * LANES, LANES)
            rows_v[r, sl] = rows_v[r, sl] * nb
        return carry2

      lax.fori_loop(0, CHUNK // LANES, group_body, 0)

      pltpu.sync_copy(rows_v, agg_sh.at[dst_v.at[j]], add=True)
      return carry

    lax.fori_loop(0, nchunk, chunk_body, 0)
    plsc.subcore_barrier()

    pltpu.sync_copy(agg_sh.at[pl.ds(row0, rows_per_tile)],
                    out_hbm.at[c].at[pl.ds(row0, rows_per_tile)])

  return seg_kernel(x, src_p, dst_p, norm_p, zeros)


def _tc_dense(partials, W_rgc, W_u, W_i, n_nodes, num_users, d, out_dim):
  rows = 1000
  grid = n_nodes // rows
  user_blocks = num_users // rows

  def body(p_ref, w1_ref, wu_ref, wi_ref, out_ref):
    agg = p_ref[0] + p_ref[1]
    h = jnp.maximum(
        jnp.dot(agg, w1_ref[...], preferred_element_type=jnp.float32), 0.0)
    u = jnp.dot(h, wu_ref[...], preferred_element_type=jnp.float32)
    v = jnp.dot(h, wi_ref[...], preferred_element_type=jnp.float32)
    sel = pl.program_id(0) < user_blocks
    out_ref[...] = jnp.maximum(jnp.where(sel, u, v), 0.0)

  h = W_rgc.shape[1]
  return pl.pallas_call(
      body,
      grid=(grid,),
      in_specs=[
          pl.BlockSpec((2, rows, d), lambda i: (0, i, 0)),
          pl.BlockSpec((d, h), lambda i: (0, 0)),
          pl.BlockSpec((h, out_dim), lambda i: (0, 0)),
          pl.BlockSpec((h, out_dim), lambda i: (0, 0)),
      ],
      out_specs=pl.BlockSpec((rows, out_dim), lambda i: (i, 0)),
      out_shape=jax.ShapeDtypeStruct((n_nodes, out_dim), jnp.float32),
  )(partials, W_rgc, W_u, W_i)


def kernel(x, edge_index, edge_norm, W_rgc, W_u, W_i):
  n_nodes, d = x.shape
  e = edge_index.shape[1]
  num_users = 2000
  out_dim = W_u.shape[1]

  nchunk = -(-e // (NW * CHUNK))
  e_pad = NW * nchunk * CHUNK
  pad = e_pad - e

  src = edge_index[0]
  dst = edge_index[1]
  src_p = jnp.concatenate(
      [src, jnp.zeros((pad,), jnp.int32)]).reshape(NW, nchunk, CHUNK)
  dst_p = jnp.concatenate(
      [dst, jnp.zeros((pad,), jnp.int32)]).reshape(NW, nchunk, CHUNK)
  norm_p = jnp.concatenate([edge_norm, jnp.zeros((pad,), jnp.float32)])
  n_pad = -(-n_nodes // (8 * NS)) * (8 * NS)
  zeros = jnp.zeros((n_pad, d), jnp.float32)

  partials = _sc_segment_sum(x, src_p, dst_p, norm_p, zeros,
                             n_pad, d, nchunk)
  out = _tc_dense(partials, W_rgc, W_u, W_i, n_nodes, num_users, d, out_dim)
  return (out[:num_users], out[num_users:])
